# baseline (speedup 1.0000x reference)
import functools

import jax
import jax.numpy as jnp
from jax import lax
from jax.experimental import pallas as pl
from jax.experimental.pallas import tpu as pltpu
from jax.experimental.pallas import tpu_sc as plsc

_B, _N, _D = 64, 576, 768
_NW = 32
_RPW = _N // _NW
_LANES = 16


def _sc_body(patch_hbm, pos_hbm, out_hbm,
             pos_v, in0, in1, out0, out1, si0, si1, so0, so1):
    w = lax.axis_index("s") * 2 + lax.axis_index("c")
    base = w * _RPW

    pltpu.sync_copy(pos_hbm.at[pl.ds(base, _RPW), :], pos_v)

    in_bufs = (in0, in1)
    out_bufs = (out0, out1)
    in_sems = (si0, si1)
    out_sems = (so0, so1)

    def in_copy(b, j):
        return pltpu.make_async_copy(
            patch_hbm.at[b, pl.ds(base, _RPW), :], in_bufs[j], in_sems[j])

    def out_copy(b, j):
        return pltpu.make_async_copy(
            out_bufs[j], out_hbm.at[b, pl.ds(base, _RPW), :], out_sems[j])

    in_copy(0, 0).start()
    in_copy(1, 1).start()

    def pair(p, carry):
        for j in range(2):
            b = p * 2 + j
            in_copy(b, j).wait()

            @pl.when(p >= 1)
            def _():
                out_copy(b - 2, j).wait()

            def row(r, c):
                for i in range(_D // _LANES):
                    sl = pl.ds(i * _LANES, _LANES)
                    out_bufs[j][r, sl] = in_bufs[j][r, sl] + pos_v[r, sl]
                return c

            lax.fori_loop(0, _RPW, row, 0)

            @pl.when(p < 31)
            def _():
                in_copy(b + 2, j).start()

            out_copy(b, j).start()
        return carry

    lax.fori_loop(0, _B // 2, pair, 0)
    out_copy(_B - 2, 0).wait()
    out_copy(_B - 1, 1).wait()


def kernel(patch, pos_table):
    mesh = plsc.VectorSubcoreMesh(core_axis_name="c", subcore_axis_name="s")
    run = functools.partial(
        pl.kernel,
        mesh=mesh,
        out_type=jax.ShapeDtypeStruct((_B, _N, _D), jnp.float32),
        scratch_types=[
            pltpu.VMEM((_RPW, _D), jnp.float32),
            pltpu.VMEM((_RPW, _D), jnp.float32),
            pltpu.VMEM((_RPW, _D), jnp.float32),
            pltpu.VMEM((_RPW, _D), jnp.float32),
            pltpu.VMEM((_RPW, _D), jnp.float32),
            pltpu.SemaphoreType.DMA,
            pltpu.SemaphoreType.DMA,
            pltpu.SemaphoreType.DMA,
            pltpu.SemaphoreType.DMA,
        ],
        compiler_params=pltpu.CompilerParams(use_tc_tiling_on_sc=False),
    )(_sc_body)
    return run(patch, pos_table)


# speedup vs baseline: 4.4409x; 4.4409x over previous
import jax
import jax.numpy as jnp
from jax.experimental import pallas as pl
from jax.experimental.pallas import tpu as pltpu


def _add_kernel(patch_ref, pos_ref, out_ref):
    out_ref[...] = patch_ref[...] + pos_ref[...]


def kernel(patch, pos_table):
    B, N, D = patch.shape
    CB = 8
    CN = 288
    return pl.pallas_call(
        _add_kernel,
        grid=(B // CB, N // CN),
        in_specs=[
            pl.BlockSpec((CB, CN, D), lambda b, n: (b, n, 0)),
            pl.BlockSpec((CN, D), lambda b, n: (n, 0)),
        ],
        out_specs=pl.BlockSpec((CB, CN, D), lambda b, n: (b, n, 0)),
        out_shape=jax.ShapeDtypeStruct((B, N, D), patch.dtype),
        compiler_params=pltpu.CompilerParams(
            dimension_semantics=("arbitrary", "arbitrary"),
            vmem_limit_bytes=128 * 1024 * 1024,
        ),
    )(patch, pos_table)


# speedup vs baseline: 4.7633x; 1.0726x over previous
import jax
import jax.numpy as jnp
from jax.experimental import pallas as pl
from jax.experimental.pallas import tpu as pltpu


def _make_body(B, N, D, CBM, NBUF):
    NCH = B // CBM

    def body(patch_hbm, pos_hbm, out_hbm, posb, inb, outb,
             possem, insem, outsem):
        def in_cp(c):
            j = c % NBUF
            return pltpu.make_async_copy(
                patch_hbm.at[pl.ds(c * CBM, CBM)], inb.at[j], insem.at[j])

        def out_cp(c):
            j = c % NBUF
            return pltpu.make_async_copy(
                outb.at[j], out_hbm.at[pl.ds(c * CBM, CBM)], outsem.at[j])

        pltpu.make_async_copy(pos_hbm, posb, possem).start()
        for c in range(NBUF):
            in_cp(c).start()
        pltpu.make_async_copy(pos_hbm, posb, possem).wait()

        for c in range(NCH):
            j = c % NBUF
            in_cp(c).wait()
            if c >= NBUF:
                out_cp(c - NBUF).wait()
            outb[j] = inb[j] + posb[...]
            if c + NBUF < NCH:
                in_cp(c + NBUF).start()
            out_cp(c).start()

        for c in range(NCH - NBUF, NCH):
            out_cp(c).wait()

    return body


def kernel(patch, pos_table):
    B, N, D = patch.shape
    CBM = 2
    NBUF = 4
    return pl.pallas_call(
        _make_body(B, N, D, CBM, NBUF),
        in_specs=[
            pl.BlockSpec(memory_space=pl.ANY),
            pl.BlockSpec(memory_space=pl.ANY),
        ],
        out_specs=pl.BlockSpec(memory_space=pl.ANY),
        out_shape=jax.ShapeDtypeStruct((B, N, D), patch.dtype),
        scratch_shapes=[
            pltpu.VMEM((N, D), patch.dtype),
            pltpu.VMEM((NBUF, CBM, N, D), patch.dtype),
            pltpu.VMEM((NBUF, CBM, N, D), patch.dtype),
            pltpu.SemaphoreType.DMA,
            pltpu.SemaphoreType.DMA((NBUF,)),
            pltpu.SemaphoreType.DMA((NBUF,)),
        ],
    )(patch, pos_table)


# speedup vs baseline: 4.7844x; 1.0044x over previous
import jax
import jax.numpy as jnp
from jax.experimental import pallas as pl
from jax.experimental.pallas import tpu as pltpu


def _make_body(B, N, D, CBM, NBUF):
    NCH = B // CBM

    def body(patch_hbm, pos_hbm, out_hbm, posb, inb, outb,
             possem, insem, outsem):
        def in_cp(c):
            j = c % NBUF
            return pltpu.make_async_copy(
                patch_hbm.at[pl.ds(c * CBM, CBM)], inb.at[j], insem.at[j])

        def out_cp(c):
            j = c % NBUF
            return pltpu.make_async_copy(
                outb.at[j], out_hbm.at[pl.ds(c * CBM, CBM)], outsem.at[j])

        pltpu.make_async_copy(pos_hbm, posb, possem).start()
        for c in range(NBUF):
            in_cp(c).start()
        pltpu.make_async_copy(pos_hbm, posb, possem).wait()

        for c in range(NCH):
            j = c % NBUF
            in_cp(c).wait()
            if c >= NBUF:
                out_cp(c - NBUF).wait()
            outb[j] = inb[j] + posb[...]
            if c + NBUF < NCH:
                in_cp(c + NBUF).start()
            out_cp(c).start()

        for c in range(NCH - NBUF, NCH):
            out_cp(c).wait()

    return body


def kernel(patch, pos_table):
    B, N, D = patch.shape
    CBM = 2
    NBUF = 6
    return pl.pallas_call(
        _make_body(B, N, D, CBM, NBUF),
        in_specs=[
            pl.BlockSpec(memory_space=pl.ANY),
            pl.BlockSpec(memory_space=pl.ANY),
        ],
        out_specs=pl.BlockSpec(memory_space=pl.ANY),
        out_shape=jax.ShapeDtypeStruct((B, N, D), patch.dtype),
        scratch_shapes=[
            pltpu.VMEM((N, D), patch.dtype),
            pltpu.VMEM((NBUF, CBM, N, D), patch.dtype),
            pltpu.VMEM((NBUF, CBM, N, D), patch.dtype),
            pltpu.SemaphoreType.DMA,
            pltpu.SemaphoreType.DMA((NBUF,)),
            pltpu.SemaphoreType.DMA((NBUF,)),
        ],
    )(patch, pos_table)
